# Initial kernel scaffold; baseline (speedup 1.0000x reference)
import jax
import jax.numpy as jnp
from jax.experimental import pallas as pl


def kernel(pred_boxes, pred_logits, boxes, labels, prior_cxcy):
    raise NotImplementedError("write your pallas kernel here")


# speedup vs baseline: 8.4892x; 8.4892x over previous
import functools

import jax
import jax.numpy as jnp
from jax import lax
from jax.experimental import pallas as pl
from jax.experimental.pallas import tpu as pltpu

_THRESHOLD = 0.5
_NEG_POS_RATIO = 3
_ALPHA = 1.0


def _loss_body(logits_ref, pbt_ref, boxes_ref, labels_ref, priort_ref, acc_ref):
    b = pl.program_id(0)
    P = priort_ref.shape[1]
    G = boxes_ref.shape[1]
    C = logits_ref.shape[2]

    f32 = jnp.float32
    i32 = jnp.int32

    pr = priort_ref[...]
    pcx, pcy, pw, ph = pr[0], pr[1], pr[2], pr[3]
    px1 = pcx - pw * 0.5
    py1 = pcy - ph * 0.5
    px2 = pcx + pw * 0.5
    py2 = pcy + ph * 0.5
    parea = pw * ph

    bb = boxes_ref[0]
    bx1 = bb[:, 0:1]
    by1 = bb[:, 1:2]
    bx2 = bb[:, 2:3]
    by2 = bb[:, 3:4]
    barea = (bx2 - bx1) * (by2 - by1)

    ltx = jnp.maximum(px1[None, :], bx1)
    lty = jnp.maximum(py1[None, :], by1)
    rbx = jnp.minimum(px2[None, :], bx2)
    rby = jnp.minimum(py2[None, :], by2)
    iw = jnp.maximum(rbx - ltx, 0.0)
    ih = jnp.maximum(rby - lty, 0.0)
    inter = iw * ih
    iou = inter / (parea[None, :] + barea - inter + 1e-10)

    giota = lax.broadcasted_iota(i32, (G, P), 0)
    piota_row = lax.broadcasted_iota(i32, (G, P), 1)
    piota = lax.broadcasted_iota(i32, (P,), 0)

    ov_best = jnp.max(iou, axis=0)
    idx_best = jnp.min(jnp.where(iou == ov_best[None, :], giota, G), axis=0)

    m_g = jnp.max(iou, axis=1, keepdims=True)
    idx_pg = jnp.min(jnp.where(iou == m_g, piota_row, P), axis=1,
                     keepdims=True)

    match = idx_pg == piota[None, :]
    forced_g = jnp.max(jnp.where(match, giota, -1), axis=0)
    forced = forced_g >= 0
    idx_best = jnp.where(forced, forced_g, idx_best)
    ov_best = jnp.where(forced, 1.0, ov_best)

    sel = giota == idx_best[None, :]
    lab_col = labels_ref[0]
    lbl = jnp.sum(jnp.where(sel, lab_col, 0), axis=0)
    lbl = jnp.where(ov_best < _THRESHOLD, 0, lbl)

    mbx1 = jnp.sum(jnp.where(sel, bx1, 0.0), axis=0)
    mby1 = jnp.sum(jnp.where(sel, by1, 0.0), axis=0)
    mbx2 = jnp.sum(jnp.where(sel, bx2, 0.0), axis=0)
    mby2 = jnp.sum(jnp.where(sel, by2, 0.0), axis=0)

    mcx = (mbx1 + mbx2) * 0.5
    mcy = (mby1 + mby2) * 0.5
    mw = mbx2 - mbx1
    mh = mby2 - mby1
    tx = (mcx - pcx) / (pw / 10.0)
    ty = (mcy - pcy) / (ph / 10.0)
    tw = jnp.log(mw / pw) * 5.0
    th = jnp.log(mh / ph) * 5.0

    pos = lbl != 0
    npos = jnp.sum(pos.astype(i32))

    pbt = pbt_ref[0]
    bsum = f32(0.0)
    for c, t in enumerate((tx, ty, tw, th)):
        d = jnp.abs(pbt[c] - t)
        sl1 = jnp.where(d < 1.0, 0.5 * d * d, d - 0.5)
        bsum = bsum + jnp.sum(jnp.where(pos, sl1, 0.0))

    x = logits_ref[0]
    m = jnp.max(x, axis=1, keepdims=True)
    s = jnp.sum(jnp.exp(x - m), axis=1, keepdims=True)
    lse = jnp.log(s) + m
    lbl_col = lbl.reshape(P, 1)
    oh = lax.broadcasted_iota(i32, (P, C), 1) == lbl_col
    xg = jnp.sum(jnp.where(oh, x, 0.0), axis=1, keepdims=True)
    ce = (lse - xg)[:, 0]

    pos_sum = jnp.sum(jnp.where(pos, ce, 0.0))

    neg = jnp.where(pos, 0.0, ce)
    bits = lax.bitcast_convert_type(neg, i32)
    kk = jnp.minimum(npos * _NEG_POS_RATIO, P)

    def bs_body(_, carry):
        lo, hi = carry
        mid = lo + (hi - lo + 1) // 2
        cnt = jnp.sum((bits >= mid).astype(i32))
        take = cnt >= kk
        return jnp.where(take, mid, lo), jnp.where(take, hi, mid - 1)

    lo, _ = lax.fori_loop(0, 31, bs_body, (i32(0), i32(0x7F800000)))
    gt = bits > lo
    cnt_gt = jnp.sum(gt.astype(i32))
    sum_gt = jnp.sum(jnp.where(gt, neg, 0.0))
    vf = lax.bitcast_convert_type(lo, f32)
    neg_sum = jnp.where(kk > 0,
                        sum_gt + (kk - cnt_gt).astype(f32) * vf,
                        0.0)

    li = lax.broadcasted_iota(i32, (8, 128), 1)
    si = lax.broadcasted_iota(i32, (8, 128), 0)
    row0 = si == 0
    upd = (jnp.where(row0 & (li == 0), npos.astype(f32), 0.0)
           + jnp.where(row0 & (li == 1), bsum, 0.0)
           + jnp.where(row0 & (li == 2), pos_sum, 0.0)
           + jnp.where(row0 & (li == 3), neg_sum, 0.0))

    @pl.when(b == 0)
    def _():
        acc_ref[...] = jnp.zeros((8, 128), f32)

    acc_ref[...] += upd


@jax.jit
def kernel(pred_boxes, pred_logits, boxes, labels, prior_cxcy):
    B, P, C = pred_logits.shape
    G = boxes.shape[1]

    pbt = jnp.transpose(pred_boxes, (0, 2, 1))
    priort = jnp.transpose(prior_cxcy, (1, 0))
    labels3 = labels.astype(jnp.int32).reshape(B, G, 1)

    acc = pl.pallas_call(
        _loss_body,
        grid=(B,),
        in_specs=[
            pl.BlockSpec((1, P, C), lambda b: (b, 0, 0)),
            pl.BlockSpec((1, 4, P), lambda b: (b, 0, 0)),
            pl.BlockSpec((1, G, 4), lambda b: (b, 0, 0)),
            pl.BlockSpec((1, G, 1), lambda b: (b, 0, 0)),
            pl.BlockSpec((4, P), lambda b: (0, 0)),
        ],
        out_specs=pl.BlockSpec((8, 128), lambda b: (0, 0)),
        out_shape=jax.ShapeDtypeStruct((8, 128), jnp.float32),
    )(pred_logits, pbt, boxes, labels3, priort)

    npt = acc[0, 0]
    box_loss = acc[0, 1] / jnp.maximum(npt * 4.0, 1.0)
    cls_loss = (acc[0, 2] + acc[0, 3]) / jnp.maximum(npt, 1.0)
    return cls_loss + _ALPHA * box_loss


# speedup vs baseline: 9.2302x; 1.0873x over previous
import jax
import jax.numpy as jnp
from jax import lax
from jax.experimental import pallas as pl

_THRESHOLD = 0.5
_NEG_POS_RATIO = 3
_ALPHA = 1.0


def _match_ce_body(logits_ref, pbt_ref, boxes_ref, labels_ref, priort_ref,
                   acc_ref, cs_ref):
    b = pl.program_id(0)
    P = priort_ref.shape[1]
    G = boxes_ref.shape[1]
    C = logits_ref.shape[2]

    f32 = jnp.float32
    i32 = jnp.int32

    pr = priort_ref[...]
    pcx, pcy, pw, ph = pr[0], pr[1], pr[2], pr[3]
    px1 = pcx - pw * 0.5
    py1 = pcy - ph * 0.5
    px2 = pcx + pw * 0.5
    py2 = pcy + ph * 0.5
    parea = pw * ph

    bb = boxes_ref[0]
    bx1 = bb[:, 0:1]
    by1 = bb[:, 1:2]
    bx2 = bb[:, 2:3]
    by2 = bb[:, 3:4]
    barea = (bx2 - bx1) * (by2 - by1)

    ltx = jnp.maximum(px1[None, :], bx1)
    lty = jnp.maximum(py1[None, :], by1)
    rbx = jnp.minimum(px2[None, :], bx2)
    rby = jnp.minimum(py2[None, :], by2)
    iw = jnp.maximum(rbx - ltx, 0.0)
    ih = jnp.maximum(rby - lty, 0.0)
    inter = iw * ih
    iou = inter / (parea[None, :] + barea - inter + 1e-10)

    giota = lax.broadcasted_iota(i32, (G, P), 0)
    piota_row = lax.broadcasted_iota(i32, (G, P), 1)
    piota = lax.broadcasted_iota(i32, (P,), 0)

    ov_best = jnp.max(iou, axis=0)
    idx_best = jnp.min(jnp.where(iou == ov_best[None, :], giota, G), axis=0)

    m_g = jnp.max(iou, axis=1, keepdims=True)
    idx_pg = jnp.min(jnp.where(iou == m_g, piota_row, P), axis=1,
                     keepdims=True)

    match = idx_pg == piota[None, :]
    forced_g = jnp.max(jnp.where(match, giota, -1), axis=0)
    forced = forced_g >= 0
    idx_best = jnp.where(forced, forced_g, idx_best)
    ov_best = jnp.where(forced, 1.0, ov_best)

    sel_f = (giota == idx_best[None, :]).astype(f32)
    lab_row = labels_ref[0].astype(f32).reshape(1, G)
    vals = jnp.concatenate([bb.T, lab_row], axis=0)
    mm = jnp.dot(vals, sel_f, preferred_element_type=f32)
    mbx1, mby1, mbx2, mby2 = mm[0], mm[1], mm[2], mm[3]
    lbl = jnp.round(mm[4]).astype(i32)
    lbl = jnp.where(ov_best < _THRESHOLD, 0, lbl)

    mcx = (mbx1 + mbx2) * 0.5
    mcy = (mby1 + mby2) * 0.5
    mw = mbx2 - mbx1
    mh = mby2 - mby1
    tx = (mcx - pcx) / (pw / 10.0)
    ty = (mcy - pcy) / (ph / 10.0)
    tw = jnp.log(mw / pw) * 5.0
    th = jnp.log(mh / ph) * 5.0

    pos = lbl != 0

    pbt = pbt_ref[0]
    bsum = f32(0.0)
    for c, t in enumerate((tx, ty, tw, th)):
        d = jnp.abs(pbt[c] - t)
        sl1 = jnp.where(d < 1.0, 0.5 * d * d, d - 0.5)
        bsum = bsum + jnp.sum(jnp.where(pos, sl1, 0.0))

    x = logits_ref[0]
    m = jnp.max(x, axis=1, keepdims=True)
    s = jnp.sum(jnp.exp(x - m), axis=1, keepdims=True)
    lse = jnp.log(s) + m
    lbl_col = lbl.reshape(P, 1)
    oh = lax.broadcasted_iota(i32, (P, C), 1) == lbl_col
    xg = jnp.sum(jnp.where(oh, x, 0.0), axis=1, keepdims=True)
    ce_col = lse - xg
    cs_col = jnp.where(lbl_col != 0, -ce_col - 1.0, ce_col)
    cs_ref[0, 0, :] = cs_col[:, 0]

    li = lax.broadcasted_iota(i32, (8, 128), 1)
    si = lax.broadcasted_iota(i32, (8, 128), 0)
    upd = jnp.where((si == 0) & (li == 1), bsum, 0.0)

    @pl.when(b == 0)
    def _():
        acc_ref[...] = jnp.zeros((8, 128), jnp.float32)

    acc_ref[...] += upd


def _mine_body(cs_ref, acc_ref):
    f32 = jnp.float32
    i32 = jnp.int32
    B = cs_ref.shape[0]
    P = cs_ref.shape[2]

    cs = cs_ref[:, 0, :]
    pos = cs < 0.0
    neg = jnp.where(pos, 0.0, cs)
    ce_pos = jnp.where(pos, -(cs + 1.0), 0.0)

    npos = jnp.sum(pos.astype(i32), axis=1, keepdims=True)
    npos_total = jnp.sum(npos).astype(f32)
    pos_sum = jnp.sum(ce_pos)

    bits = lax.bitcast_convert_type(neg, i32)
    kk = jnp.minimum(npos * _NEG_POS_RATIO, P)

    def bs_body(_, carry):
        lo, hi = carry
        mid = lo + (hi - lo + 1) // 2
        cnt = jnp.sum((bits >= mid).astype(i32), axis=1, keepdims=True)
        take = cnt >= kk
        return jnp.where(take, mid, lo), jnp.where(take, hi, mid - 1)

    init = (jnp.zeros((B, 1), i32), jnp.full((B, 1), 0x7F800000, i32))
    lo, _ = lax.fori_loop(0, 31, bs_body, init)

    gt = bits > lo
    cnt_gt = jnp.sum(gt.astype(i32), axis=1, keepdims=True)
    sum_gt = jnp.sum(jnp.where(gt, neg, 0.0), axis=1, keepdims=True)
    vf = lax.bitcast_convert_type(lo, f32)
    neg_rows = jnp.where(kk > 0,
                         sum_gt + (kk - cnt_gt).astype(f32) * vf,
                         0.0)
    neg_sum = jnp.sum(neg_rows)

    li = lax.broadcasted_iota(i32, (8, 128), 1)
    si = lax.broadcasted_iota(i32, (8, 128), 0)
    row0 = si == 0
    acc_ref[...] = (jnp.where(row0 & (li == 0), npos_total, 0.0)
                    + jnp.where(row0 & (li == 2), pos_sum, 0.0)
                    + jnp.where(row0 & (li == 3), neg_sum, 0.0))


@jax.jit
def kernel(pred_boxes, pred_logits, boxes, labels, prior_cxcy):
    B, P, C = pred_logits.shape
    G = boxes.shape[1]

    pbt = jnp.transpose(pred_boxes, (0, 2, 1))
    priort = jnp.transpose(prior_cxcy, (1, 0))
    labels3 = labels.astype(jnp.int32).reshape(B, G, 1)

    acc_a, cs = pl.pallas_call(
        _match_ce_body,
        grid=(B,),
        in_specs=[
            pl.BlockSpec((1, P, C), lambda b: (b, 0, 0)),
            pl.BlockSpec((1, 4, P), lambda b: (b, 0, 0)),
            pl.BlockSpec((1, G, 4), lambda b: (b, 0, 0)),
            pl.BlockSpec((1, G, 1), lambda b: (b, 0, 0)),
            pl.BlockSpec((4, P), lambda b: (0, 0)),
        ],
        out_specs=[
            pl.BlockSpec((8, 128), lambda b: (0, 0)),
            pl.BlockSpec((1, 1, P), lambda b: (b, 0, 0)),
        ],
        out_shape=[
            jax.ShapeDtypeStruct((8, 128), jnp.float32),
            jax.ShapeDtypeStruct((B, 1, P), jnp.float32),
        ],
    )(pred_logits, pbt, boxes, labels3, priort)

    acc_b = pl.pallas_call(
        _mine_body,
        out_shape=jax.ShapeDtypeStruct((8, 128), jnp.float32),
    )(cs)

    npt = acc_b[0, 0]
    box_loss = acc_a[0, 1] / jnp.maximum(npt * 4.0, 1.0)
    cls_loss = (acc_b[0, 2] + acc_b[0, 3]) / jnp.maximum(npt, 1.0)
    return cls_loss + _ALPHA * box_loss


# speedup vs baseline: 16.2249x; 1.7578x over previous
import jax
import jax.numpy as jnp
from jax import lax
from jax.experimental import pallas as pl

_THRESHOLD = 0.5
_NEG_POS_RATIO = 3
_ALPHA = 1.0


def _match_ce_body(logits_ref, pbt_ref, boxes_ref, labels_ref, priort_ref,
                   acc_ref, cs_ref):
    b = pl.program_id(0)
    P = priort_ref.shape[1]
    G = boxes_ref.shape[1]
    C = logits_ref.shape[2]

    f32 = jnp.float32
    i32 = jnp.int32

    pr = priort_ref[...]
    pcx, pcy, pw, ph = pr[0], pr[1], pr[2], pr[3]
    px1 = pcx - pw * 0.5
    py1 = pcy - ph * 0.5
    px2 = pcx + pw * 0.5
    py2 = pcy + ph * 0.5
    parea = pw * ph

    bb = boxes_ref[0]
    bx1 = bb[:, 0:1]
    by1 = bb[:, 1:2]
    bx2 = bb[:, 2:3]
    by2 = bb[:, 3:4]
    barea = (bx2 - bx1) * (by2 - by1)

    ltx = jnp.maximum(px1[None, :], bx1)
    lty = jnp.maximum(py1[None, :], by1)
    rbx = jnp.minimum(px2[None, :], bx2)
    rby = jnp.minimum(py2[None, :], by2)
    iw = jnp.maximum(rbx - ltx, 0.0)
    ih = jnp.maximum(rby - lty, 0.0)
    inter = iw * ih
    iou = inter / (parea[None, :] + barea - inter + 1e-10)

    giota = lax.broadcasted_iota(i32, (G, P), 0)
    piota_row = lax.broadcasted_iota(i32, (G, P), 1)
    piota = lax.broadcasted_iota(i32, (P,), 0)

    ov_best = jnp.max(iou, axis=0)
    idx_best = jnp.min(jnp.where(iou == ov_best[None, :], giota, G), axis=0)

    m_g = jnp.max(iou, axis=1, keepdims=True)
    idx_pg = jnp.min(jnp.where(iou == m_g, piota_row, P), axis=1,
                     keepdims=True)

    match = idx_pg == piota[None, :]
    forced_g = jnp.max(jnp.where(match, giota, -1), axis=0)
    forced = forced_g >= 0
    idx_best = jnp.where(forced, forced_g, idx_best)
    ov_best = jnp.where(forced, 1.0, ov_best)

    sel_f = (giota == idx_best[None, :]).astype(f32)
    lab_row = labels_ref[0].astype(f32).reshape(1, G)
    vals = jnp.concatenate([bb.T, lab_row], axis=0)
    mm = jnp.dot(vals, sel_f, preferred_element_type=f32)
    mbx1, mby1, mbx2, mby2 = mm[0], mm[1], mm[2], mm[3]
    lbl = jnp.round(mm[4]).astype(i32)
    lbl = jnp.where(ov_best < _THRESHOLD, 0, lbl)

    mcx = (mbx1 + mbx2) * 0.5
    mcy = (mby1 + mby2) * 0.5
    mw = mbx2 - mbx1
    mh = mby2 - mby1
    tx = (mcx - pcx) / (pw / 10.0)
    ty = (mcy - pcy) / (ph / 10.0)
    tw = jnp.log(mw / pw) * 5.0
    th = jnp.log(mh / ph) * 5.0

    pos = lbl != 0

    pbt = pbt_ref[0]
    bsum = f32(0.0)
    for c, t in enumerate((tx, ty, tw, th)):
        d = jnp.abs(pbt[c] - t)
        sl1 = jnp.where(d < 1.0, 0.5 * d * d, d - 0.5)
        bsum = bsum + jnp.sum(jnp.where(pos, sl1, 0.0))

    xt = logits_ref[0].T
    m = jnp.max(xt, axis=0)
    s = jnp.sum(jnp.exp(xt - m[None, :]), axis=0)
    lse = jnp.log(s) + m
    oh = lax.broadcasted_iota(i32, (C, P), 0) == lbl[None, :]
    xg = jnp.sum(jnp.where(oh, xt, 0.0), axis=0)
    ce = lse - xg
    cs_ref[0, 0, :] = jnp.where(pos, -ce - 1.0, ce)

    li = lax.broadcasted_iota(i32, (8, 128), 1)
    si = lax.broadcasted_iota(i32, (8, 128), 0)
    upd = jnp.where((si == 0) & (li == 1), bsum, 0.0)

    @pl.when(b == 0)
    def _():
        acc_ref[...] = jnp.zeros((8, 128), jnp.float32)

    acc_ref[...] += upd


def _mine_body(cs_ref, acc_ref):
    f32 = jnp.float32
    i32 = jnp.int32
    B = cs_ref.shape[0]
    P = cs_ref.shape[2]

    cs = cs_ref[:, 0, :]
    pos = cs < 0.0
    neg = jnp.where(pos, 0.0, cs)
    ce_pos = jnp.where(pos, -(cs + 1.0), 0.0)

    npos = jnp.sum(pos.astype(i32), axis=1, keepdims=True)
    npos_total = jnp.sum(npos).astype(f32)
    pos_sum = jnp.sum(ce_pos)

    bits = lax.bitcast_convert_type(neg, i32)
    kk = jnp.minimum(npos * _NEG_POS_RATIO, P)

    def bs_body(_, carry):
        lo, hi = carry
        mid = lo + (hi - lo + 1) // 2
        cnt = jnp.sum((bits >= mid).astype(i32), axis=1, keepdims=True)
        take = cnt >= kk
        return jnp.where(take, mid, lo), jnp.where(take, hi, mid - 1)

    init = (jnp.zeros((B, 1), i32), jnp.full((B, 1), 0x7F800000, i32))
    lo, _ = lax.fori_loop(0, 31, bs_body, init)

    gt = bits > lo
    cnt_gt = jnp.sum(gt.astype(i32), axis=1, keepdims=True)
    sum_gt = jnp.sum(jnp.where(gt, neg, 0.0), axis=1, keepdims=True)
    vf = lax.bitcast_convert_type(lo, f32)
    neg_rows = jnp.where(kk > 0,
                         sum_gt + (kk - cnt_gt).astype(f32) * vf,
                         0.0)
    neg_sum = jnp.sum(neg_rows)

    li = lax.broadcasted_iota(i32, (8, 128), 1)
    si = lax.broadcasted_iota(i32, (8, 128), 0)
    row0 = si == 0
    acc_ref[...] = (jnp.where(row0 & (li == 0), npos_total, 0.0)
                    + jnp.where(row0 & (li == 2), pos_sum, 0.0)
                    + jnp.where(row0 & (li == 3), neg_sum, 0.0))


@jax.jit
def kernel(pred_boxes, pred_logits, boxes, labels, prior_cxcy):
    B, P, C = pred_logits.shape
    G = boxes.shape[1]

    pbt = jnp.transpose(pred_boxes, (0, 2, 1))
    priort = jnp.transpose(prior_cxcy, (1, 0))
    labels3 = labels.astype(jnp.int32).reshape(B, G, 1)

    acc_a, cs = pl.pallas_call(
        _match_ce_body,
        grid=(B,),
        in_specs=[
            pl.BlockSpec((1, P, C), lambda b: (b, 0, 0)),
            pl.BlockSpec((1, 4, P), lambda b: (b, 0, 0)),
            pl.BlockSpec((1, G, 4), lambda b: (b, 0, 0)),
            pl.BlockSpec((1, G, 1), lambda b: (b, 0, 0)),
            pl.BlockSpec((4, P), lambda b: (0, 0)),
        ],
        out_specs=[
            pl.BlockSpec((8, 128), lambda b: (0, 0)),
            pl.BlockSpec((1, 1, P), lambda b: (b, 0, 0)),
        ],
        out_shape=[
            jax.ShapeDtypeStruct((8, 128), jnp.float32),
            jax.ShapeDtypeStruct((B, 1, P), jnp.float32),
        ],
    )(pred_logits, pbt, boxes, labels3, priort)

    acc_b = pl.pallas_call(
        _mine_body,
        out_shape=jax.ShapeDtypeStruct((8, 128), jnp.float32),
    )(cs)

    npt = acc_b[0, 0]
    box_loss = acc_a[0, 1] / jnp.maximum(npt * 4.0, 1.0)
    cls_loss = (acc_b[0, 2] + acc_b[0, 3]) / jnp.maximum(npt, 1.0)
    return cls_loss + _ALPHA * box_loss
